# Initial kernel scaffold; baseline (speedup 1.0000x reference)
import jax
import jax.numpy as jnp
from jax.experimental import pallas as pl


def kernel(idx, params):
    raise NotImplementedError("write your pallas kernel here")


# speedup vs baseline: 1.3624x; 1.3624x over previous
import functools

import jax
import jax.numpy as jnp
from jax.experimental import pallas as pl
from jax.experimental.pallas import tpu as pltpu

V = 50304
T = 2048
C = 768
NH = 12
HD = C // NH
DFF = 4 * C
E = 8
TOPK = 2

EG = 8
RB = 1024
DT = DFF // 2
VB = 384


def _ln(x, g):
    m = jnp.mean(x, axis=-1, keepdims=True)
    v = jnp.mean((x - m) ** 2, axis=-1, keepdims=True)
    return (x - m) * jax.lax.rsqrt(v + 1e-5) * g


def _embed_body(idx_ref, *refs):
    wrows = refs[:EG]
    wpe_ref = refs[EG]
    out_ref = refs[EG + 1]
    for k in range(EG):
        out_ref[k:k + 1, :] = wrows[k][0] + wpe_ref[k:k + 1, :]


def _embed(idx, wte, wpe):
    grid = (T // EG,)
    in_specs = [
        pl.BlockSpec((1, 1, C), functools.partial(
            lambda k, i, idx_ref: (idx_ref[0, EG * i + k], 0, 0), k))
        for k in range(EG)
    ] + [pl.BlockSpec((EG, C), lambda i, idx_ref: (i, 0))]
    return pl.pallas_call(
        _embed_body,
        grid_spec=pltpu.PrefetchScalarGridSpec(
            num_scalar_prefetch=1,
            grid=grid,
            in_specs=in_specs,
            out_specs=pl.BlockSpec((EG, C), lambda i, idx_ref: (i, 0)),
        ),
        out_shape=jax.ShapeDtypeStruct((T, C), jnp.float32),
    )(idx, *([wte.reshape(V, 1, C)] * EG), wpe)


def _ln_body(x_ref, g_ref, out_ref):
    out_ref[...] = _ln(x_ref[...], g_ref[...])


def _ln_call(x, g):
    return pl.pallas_call(
        _ln_body,
        out_shape=jax.ShapeDtypeStruct((T, C), jnp.float32),
    )(x, g.reshape(1, C))


def _qkv_body(x_ref, w_ref, out_ref):
    y = jax.lax.dot_general(
        x_ref[...], w_ref[...], (((1,), (1,)), ((), ())),
        preferred_element_type=jnp.float32)
    out_ref[0] = y[:, :HD]
    out_ref[1] = y[:, HD:]


def _qkv(xn, w_attn):
    grid = (3 * NH // 2,)
    return pl.pallas_call(
        _qkv_body,
        grid=grid,
        in_specs=[
            pl.BlockSpec((T, C), lambda j: (0, 0)),
            pl.BlockSpec((2 * HD, C), lambda j: (j, 0)),
        ],
        out_specs=pl.BlockSpec((2, T, HD), lambda j: (j, 0, 0)),
        out_shape=jax.ShapeDtypeStruct((3 * NH, T, HD), jnp.float32),
    )(xn, w_attn)


def _attn_body(q_ref, k_ref, v_ref, out_ref):
    r = pl.program_id(1)
    s = jax.lax.dot_general(q_ref[0], k_ref[0], (((1,), (1,)), ((), ())),
                            preferred_element_type=jnp.float32)
    s = s * (1.0 / (HD ** 0.5))
    rows = jax.lax.broadcasted_iota(jnp.int32, (RB, T), 0) + r * RB
    cols = jax.lax.broadcasted_iota(jnp.int32, (RB, T), 1)
    s = jnp.where(rows >= cols, s, jnp.float32(-1e9))
    m = jnp.max(s, axis=-1, keepdims=True)
    p = jnp.exp(s - m)
    p = p / jnp.sum(p, axis=-1, keepdims=True)
    out_ref[0] = jnp.dot(p, v_ref[0], preferred_element_type=jnp.float32)


def _attn(qkv):
    grid = (NH, T // RB)
    return pl.pallas_call(
        _attn_body,
        grid=grid,
        in_specs=[
            pl.BlockSpec((1, RB, HD), lambda h, r: (h, r, 0)),
            pl.BlockSpec((1, T, HD), lambda h, r: (NH + h, 0, 0)),
            pl.BlockSpec((1, T, HD), lambda h, r: (2 * NH + h, 0, 0)),
        ],
        out_specs=pl.BlockSpec((1, RB, HD), lambda h, r: (h, r, 0)),
        out_shape=jax.ShapeDtypeStruct((NH, T, HD), jnp.float32),
    )(qkv, qkv, qkv)


def _proj_body(a_ref, w_ref, x_ref, out_ref):
    h = pl.program_id(0)

    @pl.when(h == 0)
    def _():
        out_ref[...] = x_ref[...]

    out_ref[...] += jnp.dot(a_ref[0], w_ref[0],
                            preferred_element_type=jnp.float32)


def _proj(a, wp_resh, x):
    grid = (NH,)
    return pl.pallas_call(
        _proj_body,
        grid=grid,
        in_specs=[
            pl.BlockSpec((1, T, HD), lambda h: (h, 0, 0)),
            pl.BlockSpec((1, HD, C), lambda h: (h, 0, 0)),
            pl.BlockSpec((T, C), lambda h: (0, 0)),
        ],
        out_specs=pl.BlockSpec((T, C), lambda h: (0, 0)),
        out_shape=jax.ShapeDtypeStruct((T, C), jnp.float32),
    )(a, wp_resh, x)


def _ln2_body(x_ref, g_ref, wg_ref, xn_ref, lg_ref):
    xn = _ln(x_ref[...], g_ref[...])
    xn_ref[...] = xn
    lg_ref[...] = jax.lax.dot_general(
        xn, wg_ref[...], (((1,), (1,)), ((), ())),
        preferred_element_type=jnp.float32)


def _ln2_router(x, g, wg_pad):
    return pl.pallas_call(
        _ln2_body,
        out_shape=(jax.ShapeDtypeStruct((T, C), jnp.float32),
                   jax.ShapeDtypeStruct((T, 128), jnp.float32)),
    )(x, g.reshape(1, C), wg_pad)


def _moe_body(x_ref, w1_ref, w2_ref, we_ref, out_ref):
    e = pl.program_id(0)
    d = pl.program_id(1)

    @pl.when(jnp.logical_and(e == 0, d == 0))
    def _():
        out_ref[...] = jnp.zeros_like(out_ref)

    x = x_ref[...]
    h = jax.lax.dot_general(x, w1_ref[0], (((1,), (1,)), ((), ())),
                            preferred_element_type=jnp.float32)
    h = 0.5 * h * (1.0 + jax.lax.erf(h * (2.0 ** -0.5)))
    y = jax.lax.dot_general(h, w2_ref[0], (((1,), (1,)), ((), ())),
                            preferred_element_type=jnp.float32)
    lane = jax.lax.broadcasted_iota(jnp.int32, (T, 128), 1)
    wcol = jnp.sum(we_ref[...] * (lane == e).astype(jnp.float32),
                   axis=1, keepdims=True)
    out_ref[...] += wcol * y


def _moe_ffn(xn, w1, w2, we128):
    grid = (E, DFF // DT)
    return pl.pallas_call(
        _moe_body,
        grid=grid,
        in_specs=[
            pl.BlockSpec((T, C), lambda e, d: (0, 0)),
            pl.BlockSpec((1, DT, C), lambda e, d: (e, d, 0)),
            pl.BlockSpec((1, C, DT), lambda e, d: (e, 0, d)),
            pl.BlockSpec((T, 128), lambda e, d: (0, 0)),
        ],
        out_specs=pl.BlockSpec((T, C), lambda e, d: (0, 0)),
        out_shape=jax.ShapeDtypeStruct((T, C), jnp.float32),
    )(xn, w1, w2, we128)


def _lm_body(x_ref, g_ref, wte_ref, out_ref):
    xn = _ln(x_ref[0], g_ref[...])
    out_ref[...] = jax.lax.dot_general(
        xn, wte_ref[...], (((1,), (1,)), ((), ())),
        preferred_element_type=jnp.float32)


def _lm_head(x, g, wte):
    grid = (V // VB,)
    return pl.pallas_call(
        _lm_body,
        grid=grid,
        in_specs=[
            pl.BlockSpec((1, 1, C), lambda i: (T - 1, 0, 0)),
            pl.BlockSpec((1, C), lambda i: (0, 0)),
            pl.BlockSpec((VB, C), lambda i: (i, 0)),
        ],
        out_specs=pl.BlockSpec((1, VB), lambda i: (0, i)),
        out_shape=jax.ShapeDtypeStruct((1, V), jnp.float32),
    )(x.reshape(T, 1, C), g.reshape(1, C), wte)


def kernel(idx, params):
    idx = idx.astype(jnp.int32)
    x = _embed(idx, params['wte'], params['wpe'])
    for lp in params['layers']:
        xn1 = _ln_call(x, lp['ln1_g'])
        a = _attn(_qkv(xn1, lp['w_attn']))
        wp_resh = lp['w_proj'].reshape(C, NH, HD).transpose(1, 2, 0)
        x = _proj(a, wp_resh, x)
        wg_pad = jnp.zeros((128, C), jnp.float32).at[:E].set(lp['w_gate'])
        xn, logits_pad = _ln2_router(x, lp['ln2_g'], wg_pad)
        logits = logits_pad[:, :E]
        top_v, top_i = jax.lax.top_k(logits, TOPK)
        scores = jax.nn.softmax(top_v, axis=-1)
        we = jnp.zeros((T, E), jnp.float32)
        we = we.at[jnp.arange(T)[:, None], top_i].add(scores)
        we128 = jnp.pad(we, ((0, 0), (0, 120)))
        x = x + _moe_ffn(xn, lp['w1'], lp['w2'], we128)
    logits = _lm_head(x, params['ln_f_g'], params['wte'])
    return logits.reshape(1, 1, V)
